# baseline (speedup 1.0000x reference)
import functools

import jax
import jax.numpy as jnp
from jax import lax
from jax.experimental import pallas as pl
from jax.experimental.pallas import tpu as pltpu
from jax.experimental.pallas import tpu_sc as plsc

B, S, H = 64, 128, 2048
N = B * S
NC, NS = 2, 16
NW = NC * NS
RG, CG = 8, 4
RPG = N // RG
W = H // CG
RC = 32
L = 16

_mesh = plsc.VectorSubcoreMesh(core_axis_name="c", subcore_axis_name="s")


@functools.partial(
    pl.kernel,
    mesh=_mesh,
    out_type=jax.ShapeDtypeStruct((N, H), jnp.float32),
    scratch_types=[
        pltpu.VMEM((RPG + L,), jnp.int32),
        pltpu.VMEM((S, W), jnp.float32),
        pltpu.VMEM((RC, W), jnp.float32),
        pltpu.VMEM((RC, W), jnp.float32),
        pltpu.SemaphoreType.DMA,
        pltpu.SemaphoreType.DMA,
    ],
)
def _sc_expand(ids_hbm, tab_hbm, out_hbm,
               idx_v, tabv, buf0, buf1, ssem0, ssem1):
    wid = lax.axis_index("s") * NC + lax.axis_index("c")
    rg = wid // CG
    cg = wid % CG
    row0 = rg * RPG
    col0 = cg * W
    pltpu.sync_copy(ids_hbm.at[rg], idx_v.at[pl.ds(0, RPG)])
    pltpu.sync_copy(tab_hbm.at[:, pl.ds(col0, W)], tabv)
    bufs = (buf0, buf1)
    ssems = (ssem0, ssem1)

    def body(ci, _):
        for bi in range(2):
            rbase = ci * (2 * RC) + bi * RC

            @pl.when(ci > 0)
            def _():
                pltpu.make_async_copy(
                    bufs[bi],
                    out_hbm.at[pl.ds(row0, RC), pl.ds(col0, W)],
                    ssems[bi]).wait()

            buf = bufs[bi]

            @plsc.parallel_loop(0, RC, unroll=8)
            def _(i):
                r = idx_v[pl.ds(rbase + i, L)][0]
                for c0 in range(0, W // L, 8):
                    vals = [tabv[r, pl.ds((c0 + c) * L, L)]
                            for c in range(8)]
                    for c in range(8):
                        buf[i, pl.ds((c0 + c) * L, L)] = vals[c]

            pltpu.async_copy(
                bufs[bi],
                out_hbm.at[pl.ds(row0 + rbase, RC), pl.ds(col0, W)],
                ssems[bi])
        return 0

    lax.fori_loop(0, RPG // (2 * RC), body, 0)
    for bi in range(2):
        pltpu.make_async_copy(
            bufs[bi],
            out_hbm.at[pl.ds(row0, RC), pl.ds(col0, W)],
            ssems[bi]).wait()


RB = 1024
G = N // RB


def _tc_body(ids_ref, tab_ref, out_ref):
    ids = ids_ref[0, 0]
    onehot = (ids[:, None]
              == lax.broadcasted_iota(jnp.int32, (RB, S), 1)
              ).astype(jnp.float32)
    out_ref[...] = jnp.dot(onehot, tab_ref[...],
                           preferred_element_type=jnp.float32)


_tc_gather = pl.pallas_call(
    _tc_body,
    grid=(G,),
    in_specs=[
        pl.BlockSpec((1, 1, RB), lambda i: (i, 0, 0)),
        pl.BlockSpec((S, H), lambda i: (0, 0)),
    ],
    out_specs=pl.BlockSpec((RB, H), lambda i: (i, 0)),
    out_shape=jax.ShapeDtypeStruct((N, H), jnp.float32),
)


def kernel(prefix_ids, key_table, value_table):
    ids_sc = prefix_ids.reshape(RG, RPG)
    ids_tc = prefix_ids.reshape(G, 1, RB)
    k = _sc_expand(ids_sc, key_table)
    v = _tc_gather(ids_tc, value_table)
    return k.reshape(B, S, H), v.reshape(B, S, H)


# speedup vs baseline: 1.0401x; 1.0401x over previous
import functools

import jax
import jax.numpy as jnp
from jax import lax
from jax.experimental import pallas as pl
from jax.experimental.pallas import tpu as pltpu
from jax.experimental.pallas import tpu_sc as plsc

B, S, H = 64, 128, 2048
N = B * S
NC, NS = 2, 16
NW = NC * NS
RG, CG = 8, 4
RPG = N // RG
W = H // CG
RC = 32
L = 16

_mesh = plsc.VectorSubcoreMesh(core_axis_name="c", subcore_axis_name="s")


@functools.partial(
    pl.kernel,
    mesh=_mesh,
    out_type=jax.ShapeDtypeStruct((N, H), jnp.float32),
    scratch_types=[
        pltpu.VMEM((RPG + L,), jnp.int32),
        pltpu.VMEM((S, W), jnp.float32),
        pltpu.VMEM((RC, W), jnp.float32),
        pltpu.VMEM((RC, W), jnp.float32),
        pltpu.SemaphoreType.DMA,
        pltpu.SemaphoreType.DMA,
    ],
)
def _sc_expand(ids_hbm, tab_hbm, out_hbm,
               idx_v, tabv, buf0, buf1, ssem0, ssem1):
    wid = lax.axis_index("s") * NC + lax.axis_index("c")
    rg = wid // CG
    cg = wid % CG
    row0 = rg * RPG
    col0 = cg * W
    pltpu.sync_copy(ids_hbm.at[rg], idx_v.at[pl.ds(0, RPG)])
    pltpu.sync_copy(tab_hbm.at[:, pl.ds(col0, W)], tabv)
    bufs = (buf0, buf1)
    ssems = (ssem0, ssem1)

    def body(ci, _):
        rbase = ci * RC
        for g in range(RC // L):
            rows = idx_v[pl.ds(rbase + g * L, L)]
            for k in range(L):
                r = rows[k]
                pltpu.async_copy(
                    tabv.at[r],
                    out_hbm.at[row0 + rbase + g * L + k, pl.ds(col0, W)],
                    ssems[0])
        @pl.when(ci > 0)
        def _():
            for _k in range(RC):
                pltpu.make_async_copy(
                    tabv.at[0],
                    out_hbm.at[row0, pl.ds(col0, W)],
                    ssems[0]).wait()
        return 0

    lax.fori_loop(0, RPG // RC, body, 0)
    for _k in range(RC):
        pltpu.make_async_copy(
            tabv.at[0],
            out_hbm.at[row0, pl.ds(col0, W)],
            ssems[0]).wait()


RB = 1024
G = N // RB


def _tc_body(ids_ref, tab_ref, out_ref):
    ids = ids_ref[0, 0]
    onehot = (ids[:, None]
              == lax.broadcasted_iota(jnp.int32, (RB, S), 1)
              ).astype(jnp.float32)
    out_ref[...] = jnp.dot(onehot, tab_ref[...],
                           preferred_element_type=jnp.float32)


_tc_gather = pl.pallas_call(
    _tc_body,
    grid=(G,),
    in_specs=[
        pl.BlockSpec((1, 1, RB), lambda i: (i, 0, 0)),
        pl.BlockSpec((S, H), lambda i: (0, 0)),
    ],
    out_specs=pl.BlockSpec((RB, H), lambda i: (i, 0)),
    out_shape=jax.ShapeDtypeStruct((N, H), jnp.float32),
)


def kernel(prefix_ids, key_table, value_table):
    ids_sc = prefix_ids.reshape(RG, RPG)
    ids_tc = prefix_ids.reshape(G, 1, RB)
    k = _sc_expand(ids_sc, key_table)
    v = _tc_gather(ids_tc, value_table)
    return k.reshape(B, S, H), v.reshape(B, S, H)


# speedup vs baseline: 1.0484x; 1.0080x over previous
import functools

import jax
import jax.numpy as jnp
from jax import lax
from jax.experimental import pallas as pl
from jax.experimental.pallas import tpu as pltpu
from jax.experimental.pallas import tpu_sc as plsc

B, S, H = 64, 128, 2048
N = B * S
NC, NS = 2, 16
NW = NC * NS
RPW = N // NW
RC = 32
L = 16

_mesh = plsc.VectorSubcoreMesh(core_axis_name="c", subcore_axis_name="s")


@functools.partial(
    pl.kernel,
    mesh=_mesh,
    out_type=jax.ShapeDtypeStruct((N, H), jnp.float32),
    scratch_types=[
        pltpu.VMEM((RPW + L,), jnp.int32),
        pltpu.VMEM_SHARED((S, H), jnp.float32),
        pltpu.SemaphoreType.DMA,
    ],
)
def _sc_expand(ids_hbm, tab_hbm, out_hbm, idx_v, tabs, sem):
    wid = lax.axis_index("s") * NC + lax.axis_index("c")
    sid = lax.axis_index("s")
    row0 = wid * RPW
    rows_stage = S // NS
    pltpu.sync_copy(tab_hbm.at[pl.ds(sid * rows_stage, rows_stage)],
                    tabs.at[pl.ds(sid * rows_stage, rows_stage)])
    pltpu.sync_copy(ids_hbm.at[wid], idx_v.at[pl.ds(0, RPW)])
    plsc.subcore_barrier()

    def body(ci, _):
        rbase = ci * RC
        for g in range(RC // L):
            rows = idx_v[pl.ds(rbase + g * L, L)]
            for k in range(L):
                r = rows[k]
                pltpu.async_copy(
                    tabs.at[r],
                    out_hbm.at[row0 + rbase + g * L + k],
                    sem)
        @pl.when(ci > 0)
        def _():
            for _k in range(RC):
                pltpu.make_async_copy(
                    tabs.at[0], out_hbm.at[row0], sem).wait()
        return 0

    lax.fori_loop(0, RPW // RC, body, 0)
    for _k in range(RC):
        pltpu.make_async_copy(
            tabs.at[0], out_hbm.at[row0], sem).wait()


RB = 1024
G = N // RB


def _tc_body(ids_ref, tab_ref, out_ref):
    ids = ids_ref[0, 0]
    onehot = (ids[:, None]
              == lax.broadcasted_iota(jnp.int32, (RB, S), 1)
              ).astype(jnp.float32)
    out_ref[...] = jnp.dot(onehot, tab_ref[...],
                           preferred_element_type=jnp.float32)


_tc_gather = pl.pallas_call(
    _tc_body,
    grid=(G,),
    in_specs=[
        pl.BlockSpec((1, 1, RB), lambda i: (i, 0, 0)),
        pl.BlockSpec((S, H), lambda i: (0, 0)),
    ],
    out_specs=pl.BlockSpec((RB, H), lambda i: (i, 0)),
    out_shape=jax.ShapeDtypeStruct((N, H), jnp.float32),
)


def kernel(prefix_ids, key_table, value_table):
    ids_sc = prefix_ids.reshape(NW, RPW)
    ids_tc = prefix_ids.reshape(G, 1, RB)
    k = _sc_expand(ids_sc, key_table)
    v = _tc_gather(ids_tc, value_table)
    return k.reshape(B, S, H), v.reshape(B, S, H)


# speedup vs baseline: 1.0502x; 1.0018x over previous
import functools

import jax
import jax.numpy as jnp
from jax import lax
from jax.experimental import pallas as pl
from jax.experimental.pallas import tpu as pltpu
from jax.experimental.pallas import tpu_sc as plsc

B, S, H = 64, 128, 2048
N = B * S
NC, NS = 2, 16
NW = NC * NS
RPW = N // NW
RC = 32
L = 16

_mesh = plsc.VectorSubcoreMesh(core_axis_name="c", subcore_axis_name="s")


@functools.partial(
    pl.kernel,
    mesh=_mesh,
    out_type=jax.ShapeDtypeStruct((N, H), jnp.float32),
    scratch_types=[
        pltpu.VMEM((RPW + L,), jnp.int32),
        pltpu.VMEM_SHARED((S, H), jnp.float32),
        pltpu.SemaphoreType.DMA,
    ],
)
def _sc_expand(ids_hbm, tab_hbm, out_hbm, idx_v, tabs, sem):
    wid = lax.axis_index("s") * NC + lax.axis_index("c")
    sid = lax.axis_index("s")
    row0 = wid * RPW
    rows_stage = S // NS
    pltpu.sync_copy(tab_hbm.at[pl.ds(sid * rows_stage, rows_stage)],
                    tabs.at[pl.ds(sid * rows_stage, rows_stage)])
    pltpu.sync_copy(ids_hbm.at[wid], idx_v.at[pl.ds(0, RPW)])
    plsc.subcore_barrier()

    def body(ci, _):
        rbase = ci * RC
        for g in range(RC // L):
            rows = idx_v[pl.ds(rbase + g * L, L)]
            for k in range(L):
                r = rows[k]
                pltpu.async_copy(
                    tabs.at[r],
                    out_hbm.at[row0 + rbase + g * L + k],
                    sem)
        @pl.when(ci > 0)
        def _():
            for _k in range(RC):
                pltpu.make_async_copy(
                    tabs.at[0], out_hbm.at[row0], sem).wait()
        return 0

    lax.fori_loop(0, RPW // RC, body, 0)
    for _k in range(RC):
        pltpu.make_async_copy(
            tabs.at[0], out_hbm.at[row0], sem).wait()


RB = 1024
G = N // RB


def _tc_body(ids_ref, tab_ref, out_ref):
    ids = ids_ref[0, 0]
    onehot = (ids[:, None]
              == lax.broadcasted_iota(jnp.int32, (RB, S), 1)
              ).astype(jnp.float32)
    out_ref[...] = jnp.dot(onehot, tab_ref[...],
                           preferred_element_type=jnp.float32)


_tc_gather = pl.pallas_call(
    _tc_body,
    grid=(G,),
    in_specs=[
        pl.BlockSpec((1, 1, RB), lambda i: (i, 0, 0)),
        pl.BlockSpec((S, H), lambda i: (0, 0)),
    ],
    out_specs=pl.BlockSpec((RB, H), lambda i: (i, 0)),
    out_shape=jax.ShapeDtypeStruct((N, H), jnp.float32),
)


def kernel(prefix_ids, key_table, value_table):
    ids_sc = prefix_ids.reshape(NW, RPW)
    ids_tc = prefix_ids.reshape(G, 1, RB)
    v = _tc_gather(ids_tc, value_table)
    k = _sc_expand(ids_sc, key_table)
    return k.reshape(B, S, H), v.reshape(B, S, H)
